# baseline (speedup 1.0000x reference)
import functools

import numpy as np

import jax
import jax.numpy as jnp
from jax import lax
from jax.experimental import pallas as pl
from jax.experimental.pallas import tpu as pltpu
from jax.experimental.pallas import tpu_sc as plsc

N = 10000
E = 320000
D = 128
C = 64

NC = 2
NS = 16
NW = NC * NS
EPW = E // NW
BB = 128
NB_PAD = 80
EPW_PAD = NB_PAD * BB
NPAD = 10240
_PERM = [32 * (p // 32) + (p % 32) // 2 + 16 * (p % 2) for p in range(64)]
ROWS_PER_TILE = NPAD // NS


def _proj_body(x_ref, w_ref, asv_ref, adv_ref, p_ref,
               hbf_ref, as_ref, ad_ref):
    h = jnp.dot(x_ref[...], w_ref[...], preferred_element_type=jnp.float32)
    as_ref[...] = jnp.dot(h, asv_ref[...], preferred_element_type=jnp.float32)
    ad_ref[...] = jnp.dot(h, adv_ref[...], preferred_element_type=jnp.float32)
    hbf_ref[...] = jnp.dot(
        h, p_ref[...], preferred_element_type=jnp.float32).astype(jnp.bfloat16)


def _proj(x, W, asv, adv, pmat):
    blk = 2000
    grid = N // blk
    return pl.pallas_call(
        _proj_body,
        grid=(grid,),
        in_specs=[
            pl.BlockSpec((blk, D), lambda i: (i, 0)),
            pl.BlockSpec((D, C), lambda i: (0, 0)),
            pl.BlockSpec((C, 1), lambda i: (0, 0)),
            pl.BlockSpec((C, 1), lambda i: (0, 0)),
            pl.BlockSpec((C, C), lambda i: (0, 0)),
        ],
        out_specs=[
            pl.BlockSpec((blk, C), lambda i: (i, 0)),
            pl.BlockSpec((blk, 1), lambda i: (i, 0)),
            pl.BlockSpec((blk, 1), lambda i: (i, 0)),
        ],
        out_shape=[
            jax.ShapeDtypeStruct((N, C), jnp.bfloat16),
            jax.ShapeDtypeStruct((N, 1), jnp.float32),
            jax.ShapeDtypeStruct((N, 1), jnp.float32),
        ],
    )(x, W, asv, adv, pmat)


def _edge_body(ea_ref, we_ref, aev_ref, out_ref):
    wev = jnp.dot(we_ref[...], aev_ref[...], preferred_element_type=jnp.float32)
    out_ref[...] = jnp.dot(ea_ref[...], wev, preferred_element_type=jnp.float32)


def _edge_logits(edge_attr, W_e, aev):
    blk = 20000
    grid = E // blk
    de = edge_attr.shape[1]
    return pl.pallas_call(
        _edge_body,
        grid=(grid,),
        in_specs=[
            pl.BlockSpec((blk, de), lambda i: (i, 0)),
            pl.BlockSpec((de, C), lambda i: (0, 0)),
            pl.BlockSpec((C, 1), lambda i: (0, 0)),
        ],
        out_specs=pl.BlockSpec((blk, 1), lambda i: (i, 0)),
        out_shape=jax.ShapeDtypeStruct((E, 1), jnp.float32),
    )(edge_attr, W_e, aev)


def _sc_body(ebase, h_hbm, asrc_hbm, adst_hbm, ei_hbm, ae_hbm,
             acc_out, den_out,
             asrc_v, adst_v, src_v, dst_v, ae_v, zbuf,
             rows_g, rows_s, acc_sh, den_sh, semg0, semg1, sems0, sems1,
             semd):
    cid = 0
    sid = lax.axis_index("s")
    wid = ebase + sid

    pltpu.sync_copy(asrc_hbm, asrc_v.at[pl.ds(0, N)])
    pltpu.sync_copy(adst_hbm, adst_v.at[pl.ds(0, N)])
    pltpu.sync_copy(ei_hbm.at[pl.ds(wid * EPW, EPW)], src_v.at[pl.ds(0, EPW)])
    pltpu.sync_copy(ei_hbm.at[pl.ds(E + wid * EPW, EPW)],
                    dst_v.at[pl.ds(0, EPW)])
    pltpu.sync_copy(ae_hbm.at[pl.ds(wid * EPW, EPW)], ae_v.at[pl.ds(0, EPW)])

    for t in range((EPW_PAD - EPW) // 16):
        o = EPW + t * 16
        src_v[pl.ds(o, 16)] = jnp.zeros((16,), jnp.int32)
        dst_v[pl.ds(o, 16)] = jnp.zeros((16,), jnp.int32)
        ae_v[pl.ds(o, 16)] = jnp.full((16,), -1e30, jnp.float32)

    def _zden(i, _):
        zbuf[pl.ds(i * 16, 16)] = jnp.zeros((16,), jnp.float32)
        return 0
    lax.fori_loop(0, (NPAD // NS) // 16, _zden, 0)
    pltpu.sync_copy(zbuf, den_sh.at[pl.ds(sid * (NPAD // NS), NPAD // NS)])

    def _zrow(i, _):
        for q in range(4):
            rows_s[0, i, pl.ds(q * 16, 16)] = jnp.zeros((16,), jnp.float32)
        return 0
    lax.fori_loop(0, BB, _zrow, 0)
    for kk in range(5):
        pltpu.sync_copy(
            rows_s.at[0],
            acc_sh.at[pl.ds(sid * ROWS_PER_TILE + kk * BB, BB)])

    def _s_grp(i, _):
        o = i * 16
        svec = src_v[pl.ds(o, 16)]
        dvec = dst_v[pl.ds(o, 16)]
        a = plsc.load_gather(asrc_v, [svec])
        b = plsc.load_gather(adst_v, [dvec])
        z = a + b + ae_v[pl.ds(o, 16)]
        z = jnp.where(z > 0, z, z * 0.2)
        ae_v[pl.ds(o, 16)] = jnp.exp(z)
        return 0
    lax.fori_loop(0, EPW_PAD // 16, _s_grp, 0)

    plsc.subcore_barrier()

    def _scale(j, gb, sb):
        def _grp(g, _):
            base = g * 16
            s16 = ae_v[pl.ds(j * BB + base, 16)]
            for l in range(16):
                sp = s16.at[jnp.full((16,), l, jnp.int32)].get(
                    mode="promise_in_bounds")
                e = base + l
                for q in range(2):
                    vb = rows_g[gb, e, pl.ds(q * 32, 32)]
                    f0, f1 = plsc.unpack(vb, format=plsc.PackFormat.INTERLEAVED)
                    rows_s[sb, e, pl.ds(q * 32, 16)] = f0 * sp
                    rows_s[sb, e, pl.ds(q * 32 + 16, 16)] = f1 * sp
            return 0
        lax.fori_loop(0, BB // 16, _grp, 0)

    def _wait_gather(j, gb):
        pltpu.make_async_copy(
            h_hbm.at[src_v.at[pl.ds(j * BB, BB)]], rows_g.at[gb], [semg0, semg1][gb]).wait()

    def _wait_scatter(j, sb):
        pltpu.make_async_copy(
            rows_s.at[sb], acc_sh.at[dst_v.at[pl.ds(j * BB, BB)]], [sems0, sems1][sb]).wait()

    pltpu.async_copy(h_hbm.at[src_v.at[pl.ds(0, BB)]], rows_g.at[0], semg0)
    pltpu.async_copy(h_hbm.at[src_v.at[pl.ds(BB, BB)]], rows_g.at[1], semg1)

    def _batch(i, _):
        for par in range(2):
            j = 2 * i + par
            gb = par
            sem_s = [sems0, sems1][par]
            _wait_gather(j, gb)

            @pl.when(i > 0)
            def _():
                _wait_scatter(j - 2, par)

            _scale(j, gb, par)
            pltpu.async_copy(
                rows_s.at[par], acc_sh.at[dst_v.at[pl.ds(j * BB, BB)]], sem_s, add=True)
            pltpu.async_copy(
                ae_v.at[pl.ds(j * BB, BB)], den_sh.at[dst_v.at[pl.ds(j * BB, BB)]], semd, add=True)

            @pl.when(j < NB_PAD - 2)
            def _():
                pltpu.async_copy(
                    h_hbm.at[src_v.at[pl.ds((j + 2) * BB, BB)]], rows_g.at[gb],
                    [semg0, semg1][gb])
        return 0
    lax.fori_loop(0, NB_PAD // 2, _batch, 0)
    _wait_scatter(NB_PAD - 2, 0)
    _wait_scatter(NB_PAD - 1, 1)

    def _drain_den(j, _):
        pltpu.make_async_copy(
            ae_v.at[pl.ds(j * BB, BB)], den_sh.at[dst_v.at[pl.ds(j * BB, BB)]], semd).wait()
        return 0
    lax.fori_loop(0, NB_PAD, _drain_den, 0)

    plsc.subcore_barrier()
    pltpu.sync_copy(
        acc_sh.at[pl.ds(sid * ROWS_PER_TILE, ROWS_PER_TILE)],
        acc_out.at[pl.ds(sid * ROWS_PER_TILE, ROWS_PER_TILE)])
    pltpu.sync_copy(
        den_sh.at[pl.ds(sid * (NPAD // NS), NPAD // NS)],
        den_out.at[pl.ds(sid * (NPAD // NS), NPAD // NS)])


def _sc_call_half(ebase, h, asrc, adst, ei_flat, ae_flat):
    mesh = plsc.VectorSubcoreMesh(
        core_axis_name="c", subcore_axis_name="s", num_cores=1)
    f = functools.partial(
        pl.kernel,
        mesh=mesh,
        compiler_params=pltpu.CompilerParams(
            needs_layout_passes=False, use_tc_tiling_on_sc=False),
        out_type=[
            jax.ShapeDtypeStruct((NPAD, C), jnp.float32),
            jax.ShapeDtypeStruct((NPAD,), jnp.float32),
        ],
        scratch_types=[
            pltpu.VMEM((NPAD,), jnp.float32),
            pltpu.VMEM((NPAD,), jnp.float32),
            pltpu.VMEM((EPW_PAD,), jnp.int32),
            pltpu.VMEM((EPW_PAD,), jnp.int32),
            pltpu.VMEM((EPW_PAD,), jnp.float32),
            pltpu.VMEM((NPAD // NS,), jnp.float32),
            pltpu.VMEM((2, BB, C), jnp.bfloat16),
            pltpu.VMEM((2, BB, C), jnp.float32),
            pltpu.VMEM_SHARED((NPAD, C), jnp.float32),
            pltpu.VMEM_SHARED((NPAD,), jnp.float32),
            pltpu.SemaphoreType.DMA,
            pltpu.SemaphoreType.DMA,
            pltpu.SemaphoreType.DMA,
            pltpu.SemaphoreType.DMA,
            pltpu.SemaphoreType.DMA,
        ],
    )(functools.partial(_sc_body, ebase))
    return f(h, asrc, adst, ei_flat, ae_flat)


def _fin_body(a0_ref, a1_ref, d0_ref, d1_ref, b_ref, o_ref):
    den = (d0_ref[...] + d1_ref[...])[:N]
    o_ref[...] = ((a0_ref[:N] + a1_ref[:N]) / (den[:, None] + 1e-16)
                  + b_ref[...])


def _finalize(acc0, acc1, den0, den1, bias2d):
    return pl.pallas_call(
        _fin_body,
        out_shape=jax.ShapeDtypeStruct((N, C), jnp.float32),
    )(acc0, acc1, den0, den1, bias2d)


def kernel(x, edge_index, edge_attr, W, att_src, att_dst, W_e, att_edge, bias):
    asv = att_src.reshape(C, 1)
    adv = att_dst.reshape(C, 1)
    aev = att_edge.reshape(C, 1)

    pmat = np.zeros((C, C), np.float32)
    for p in range(C):
        pmat[_PERM[p], p] = 1.0
    hbf, a_s, a_d = _proj(x, W, asv, adv, jnp.asarray(pmat))
    ae = _edge_logits(edge_attr, W_e, aev)

    args = (hbf, a_s.reshape(N), a_d.reshape(N),
            edge_index.reshape(2 * E), ae.reshape(E))
    acc0, den0 = _sc_call_half(0, *args)
    acc1, den1 = _sc_call_half(NS, *args)
    return _finalize(acc0, acc1, den0, den1, bias.reshape(1, C))


# speedup vs baseline: 1.2634x; 1.2634x over previous
import functools

import numpy as np

import jax
import jax.numpy as jnp
from jax import lax
from jax.experimental import pallas as pl
from jax.experimental.pallas import tpu as pltpu
from jax.experimental.pallas import tpu_sc as plsc

N = 10000
E = 320000
D = 128
C = 64

NC = 2
NS = 16
NW = NC * NS
EPW = E // NW
BB = 128
NB_PAD = 80
EPW_PAD = NB_PAD * BB
NPAD = 10240
_PERM = [32 * (p // 32) + (p % 32) // 2 + 16 * (p % 2) for p in range(64)]
ROWS_PER_TILE = NPAD // NS


def _proj_body(x_ref, w_ref, asv_ref, adv_ref, p_ref,
               hbf_ref, as_ref, ad_ref):
    h = jnp.dot(x_ref[...], w_ref[...], preferred_element_type=jnp.float32)
    as_ref[...] = jnp.dot(h, asv_ref[...], preferred_element_type=jnp.float32)
    ad_ref[...] = jnp.dot(h, adv_ref[...], preferred_element_type=jnp.float32)
    hbf_ref[...] = jnp.dot(
        h, p_ref[...], preferred_element_type=jnp.float32).astype(jnp.bfloat16)


def _proj(x, W, asv, adv, pmat):
    blk = 2000
    grid = N // blk
    return pl.pallas_call(
        _proj_body,
        grid=(grid,),
        in_specs=[
            pl.BlockSpec((blk, D), lambda i: (i, 0)),
            pl.BlockSpec((D, C), lambda i: (0, 0)),
            pl.BlockSpec((C, 1), lambda i: (0, 0)),
            pl.BlockSpec((C, 1), lambda i: (0, 0)),
            pl.BlockSpec((C, C), lambda i: (0, 0)),
        ],
        out_specs=[
            pl.BlockSpec((blk, C), lambda i: (i, 0)),
            pl.BlockSpec((blk, 1), lambda i: (i, 0)),
            pl.BlockSpec((blk, 1), lambda i: (i, 0)),
        ],
        out_shape=[
            jax.ShapeDtypeStruct((N, C), jnp.bfloat16),
            jax.ShapeDtypeStruct((N, 1), jnp.float32),
            jax.ShapeDtypeStruct((N, 1), jnp.float32),
        ],
    )(x, W, asv, adv, pmat)


def _edge_body(ea_ref, we_ref, aev_ref, out_ref):
    wev = jnp.dot(we_ref[...], aev_ref[...], preferred_element_type=jnp.float32)
    out_ref[...] = jnp.dot(ea_ref[...], wev, preferred_element_type=jnp.float32)


def _edge_logits(edge_attr, W_e, aev):
    blk = 20000
    grid = E // blk
    de = edge_attr.shape[1]
    return pl.pallas_call(
        _edge_body,
        grid=(grid,),
        in_specs=[
            pl.BlockSpec((blk, de), lambda i: (i, 0)),
            pl.BlockSpec((de, C), lambda i: (0, 0)),
            pl.BlockSpec((C, 1), lambda i: (0, 0)),
        ],
        out_specs=pl.BlockSpec((blk, 1), lambda i: (i, 0)),
        out_shape=jax.ShapeDtypeStruct((E, 1), jnp.float32),
    )(edge_attr, W_e, aev)


def _sc_body(h_hbm, asrc_hbm, adst_hbm, ei_hbm, ae_hbm,
             acc_out, den_out,
             asrc_v, adst_v, src_v, dst_v, ae_v, zbuf,
             rows_g, rows_s, acc_sh, den_sh, semg0, semg1, sems0, sems1,
             semd, sems2, sems3):
    cid = lax.axis_index("c")
    sid = lax.axis_index("s")
    wid = sid * NC + cid

    pltpu.sync_copy(asrc_hbm, asrc_v.at[pl.ds(0, N)])
    pltpu.sync_copy(adst_hbm, adst_v.at[pl.ds(0, N)])
    pltpu.sync_copy(ei_hbm.at[pl.ds(wid * EPW, EPW)], src_v.at[pl.ds(0, EPW)])
    pltpu.sync_copy(ei_hbm.at[pl.ds(E + wid * EPW, EPW)],
                    dst_v.at[pl.ds(0, EPW)])
    pltpu.sync_copy(ae_hbm.at[pl.ds(wid * EPW, EPW)], ae_v.at[pl.ds(0, EPW)])

    for t in range((EPW_PAD - EPW) // 16):
        o = EPW + t * 16
        src_v[pl.ds(o, 16)] = jnp.zeros((16,), jnp.int32)
        dst_v[pl.ds(o, 16)] = jnp.zeros((16,), jnp.int32)
        ae_v[pl.ds(o, 16)] = jnp.full((16,), -1e30, jnp.float32)

    def _zden(i, _):
        zbuf[pl.ds(i * 16, 16)] = jnp.zeros((16,), jnp.float32)
        return 0
    lax.fori_loop(0, (NPAD // NS) // 16, _zden, 0)
    pltpu.sync_copy(zbuf, den_sh.at[pl.ds(sid * (NPAD // NS), NPAD // NS)])

    def _zrow(i, _):
        for q in range(4):
            rows_s[0, i, pl.ds(q * 16, 16)] = jnp.zeros((16,), jnp.float32)
        return 0
    lax.fori_loop(0, BB, _zrow, 0)
    for kk in range(5):
        pltpu.sync_copy(
            rows_s.at[0],
            acc_sh.at[pl.ds(sid * ROWS_PER_TILE + kk * BB, BB)])

    def _s_grp(i, _):
        o = i * 16
        svec = src_v[pl.ds(o, 16)]
        dvec = dst_v[pl.ds(o, 16)]
        a = plsc.load_gather(asrc_v, [svec])
        b = plsc.load_gather(adst_v, [dvec])
        z = a + b + ae_v[pl.ds(o, 16)]
        z = jnp.where(z > 0, z, z * 0.2)
        ae_v[pl.ds(o, 16)] = jnp.exp(z)
        return 0
    lax.fori_loop(0, EPW_PAD // 16, _s_grp, 0)

    plsc.subcore_barrier()

    def _scale(j, gb, sb):
        def _grp(g, _):
            base = g * 16
            s16 = ae_v[pl.ds(j * BB + base, 16)]
            for l in range(16):
                sp = s16.at[jnp.full((16,), l, jnp.int32)].get(
                    mode="promise_in_bounds")
                e = base + l
                for q in range(2):
                    vb = rows_g[gb, e, pl.ds(q * 32, 32)]
                    f0, f1 = plsc.unpack(vb, format=plsc.PackFormat.INTERLEAVED)
                    rows_s[sb, e, pl.ds(q * 32, 16)] = f0 * sp
                    rows_s[sb, e, pl.ds(q * 32 + 16, 16)] = f1 * sp
            return 0
        lax.fori_loop(0, BB // 16, _grp, 0)

    HB = BB // 2

    def _issue_gather(j, gb):
        pltpu.async_copy(
            h_hbm.at[src_v.at[pl.ds(j * BB, HB)]],
            rows_g.at[gb, pl.ds(0, HB)], [semg0, semg1][gb])
        pltpu.async_copy(
            h_hbm.at[src_v.at[pl.ds(j * BB + HB, HB)]],
            rows_g.at[gb, pl.ds(HB, HB)], [sems2, sems3][gb])

    def _wait_gather(j, gb):
        pltpu.make_async_copy(
            h_hbm.at[src_v.at[pl.ds(j * BB, HB)]],
            rows_g.at[gb, pl.ds(0, HB)], [semg0, semg1][gb]).wait()
        pltpu.make_async_copy(
            h_hbm.at[src_v.at[pl.ds(j * BB + HB, HB)]],
            rows_g.at[gb, pl.ds(HB, HB)], [sems2, sems3][gb]).wait()

    def _wait_scatter(j, sb):
        pltpu.make_async_copy(
            rows_s.at[sb], acc_sh.at[dst_v.at[pl.ds(j * BB, BB)]], [sems0, sems1][sb]).wait()

    _issue_gather(0, 0)
    _issue_gather(1, 1)

    def _batch(i, _):
        for par in range(2):
            j = 2 * i + par
            gb = par
            sem_s = [sems0, sems1][par]
            _wait_gather(j, gb)

            @pl.when(i > 0)
            def _():
                _wait_scatter(j - 2, par)

            _scale(j, gb, par)
            pltpu.async_copy(
                rows_s.at[par], acc_sh.at[dst_v.at[pl.ds(j * BB, BB)]], sem_s, add=True)
            pltpu.async_copy(
                ae_v.at[pl.ds(j * BB, BB)], den_sh.at[dst_v.at[pl.ds(j * BB, BB)]], semd, add=True)

            @pl.when(j < NB_PAD - 2)
            def _():
                _issue_gather(j + 2, gb)
        return 0
    lax.fori_loop(0, NB_PAD // 2, _batch, 0)
    _wait_scatter(NB_PAD - 2, 0)
    _wait_scatter(NB_PAD - 1, 1)

    def _drain_den(j, _):
        pltpu.make_async_copy(
            ae_v.at[pl.ds(j * BB, BB)], den_sh.at[dst_v.at[pl.ds(j * BB, BB)]], semd).wait()
        return 0
    lax.fori_loop(0, NB_PAD, _drain_den, 0)

    plsc.subcore_barrier()
    pltpu.sync_copy(
        acc_sh.at[pl.ds(sid * ROWS_PER_TILE, ROWS_PER_TILE)],
        acc_out.at[cid, pl.ds(sid * ROWS_PER_TILE, ROWS_PER_TILE)])
    pltpu.sync_copy(
        den_sh.at[pl.ds(sid * (NPAD // NS), NPAD // NS)],
        den_out.at[pl.ds(cid * NPAD + sid * (NPAD // NS), NPAD // NS)])


def _sc_call(h, asrc, adst, ei_flat, ae_flat):
    mesh = plsc.VectorSubcoreMesh(core_axis_name="c", subcore_axis_name="s")
    f = functools.partial(
        pl.kernel,
        mesh=mesh,
        compiler_params=pltpu.CompilerParams(
            needs_layout_passes=False, use_tc_tiling_on_sc=False),
        out_type=[
            jax.ShapeDtypeStruct((NC, NPAD, C), jnp.float32),
            jax.ShapeDtypeStruct((NC * NPAD,), jnp.float32),
        ],
        scratch_types=[
            pltpu.VMEM((NPAD,), jnp.float32),
            pltpu.VMEM((NPAD,), jnp.float32),
            pltpu.VMEM((EPW_PAD,), jnp.int32),
            pltpu.VMEM((EPW_PAD,), jnp.int32),
            pltpu.VMEM((EPW_PAD,), jnp.float32),
            pltpu.VMEM((NPAD // NS,), jnp.float32),
            pltpu.VMEM((2, BB, C), jnp.bfloat16),
            pltpu.VMEM((2, BB, C), jnp.float32),
            pltpu.VMEM_SHARED((NPAD, C), jnp.float32),
            pltpu.VMEM_SHARED((NPAD,), jnp.float32),
            pltpu.SemaphoreType.DMA,
            pltpu.SemaphoreType.DMA,
            pltpu.SemaphoreType.DMA,
            pltpu.SemaphoreType.DMA,
            pltpu.SemaphoreType.DMA,
            pltpu.SemaphoreType.DMA,
            pltpu.SemaphoreType.DMA,
        ],
    )(_sc_body)
    return f(h, asrc, adst, ei_flat, ae_flat)


def _fin_body(acc_ref, den_ref, b_ref, o_ref):
    den = jnp.sum(den_ref[...], axis=0)[:N]
    o_ref[...] = ((acc_ref[0, :N] + acc_ref[1, :N]) / (den[:, None] + 1e-16)
                  + b_ref[...])


def _finalize(acc, den, bias2d):
    return pl.pallas_call(
        _fin_body,
        out_shape=jax.ShapeDtypeStruct((N, C), jnp.float32),
    )(acc, den, bias2d)


def kernel(x, edge_index, edge_attr, W, att_src, att_dst, W_e, att_edge, bias):
    asv = att_src.reshape(C, 1)
    adv = att_dst.reshape(C, 1)
    aev = att_edge.reshape(C, 1)

    pmat = np.zeros((C, C), np.float32)
    for p in range(C):
        pmat[_PERM[p], p] = 1.0
    hbf, a_s, a_d = _proj(x, W, asv, adv, jnp.asarray(pmat))
    ae = _edge_logits(edge_attr, W_e, aev)

    acc, den = _sc_call(hbf, a_s.reshape(N), a_d.reshape(N),
                        edge_index.reshape(2 * E), ae.reshape(E))
    return _finalize(acc, den.reshape(NC, NPAD), bias.reshape(1, C))
